# Initial kernel scaffold; baseline (speedup 1.0000x reference)
import jax
import jax.numpy as jnp
from jax.experimental import pallas as pl


def kernel(indices, features, table, W_feat, b_feat, W1, b1, W2, b2):
    raise NotImplementedError("write your pallas kernel here")


# speedup vs baseline: 1.6800x; 1.6800x over previous
import functools

import jax
import jax.numpy as jnp
from jax import lax
from jax.experimental import pallas as pl
from jax.experimental.pallas import tpu as pltpu
from jax.experimental.pallas import tpu_sc as plsc


_SC_INFO = plsc.get_sparse_core_info()
_NW = _SC_INFO.num_cores * _SC_INFO.num_subcores


@functools.lru_cache(maxsize=None)
def _make_sc_gather(V, D, B):
  b_per_w = B // _NW
  mesh = plsc.VectorSubcoreMesh(core_axis_name="c", subcore_axis_name="s")

  @functools.partial(
      pl.kernel,
      mesh=mesh,
      out_type=jax.ShapeDtypeStruct((B, D), jnp.float32),
      scratch_types=[
          pltpu.VMEM((b_per_w,), jnp.int32),
          pltpu.VMEM((b_per_w, D), jnp.float32),
          pltpu.SemaphoreType.DMA,
      ],
      name="sc_embedding_gather",
  )
  def gather_kernel(table_hbm, idx_hbm, out_hbm, idx_v, rows_v, sem):
    wid = lax.axis_index("s") * _SC_INFO.num_cores + lax.axis_index("c")
    base = wid * b_per_w
    pltpu.sync_copy(idx_hbm.at[pl.ds(base, b_per_w)], idx_v)
    pltpu.async_copy(table_hbm.at[idx_v], rows_v, sem).wait()
    pltpu.sync_copy(rows_v, out_hbm.at[pl.ds(base, b_per_w)])

  return gather_kernel


def _tc_fused_body(feat_ref, id_ref, wf_ref, bf_ref, w1a_ref, w1b_ref,
                   b1_ref, w2_ref, b2_ref, out_ref):
  idr = id_ref[...]
  fr = (jnp.dot(feat_ref[...], wf_ref[...], preferred_element_type=jnp.float32)
        + bf_ref[...])
  h = jnp.dot(idr, w1a_ref[...], preferred_element_type=jnp.float32)
  h += jnp.dot(fr, w1b_ref[...], preferred_element_type=jnp.float32)
  h = jnp.maximum(h + b1_ref[...], 0.0)
  g = jnp.dot(h, w2_ref[...], preferred_element_type=jnp.float32) + b2_ref[...]
  gate = jax.nn.sigmoid(g)
  out_ref[...] = gate * idr + (1.0 - gate) * fr


def _tc_fused(features, id_repr, W_feat, b_feat, W1a, W1b, b1, W2, b2,
              block_b=2048):
  B, F = features.shape
  D = id_repr.shape[1]
  H = W1a.shape[1]
  grid = (B // block_b,)
  full = lambda *s: pl.BlockSpec(s, lambda i: (0,) * len(s))
  return pl.pallas_call(
      _tc_fused_body,
      grid=grid,
      in_specs=[
          pl.BlockSpec((block_b, F), lambda i: (i, 0)),
          pl.BlockSpec((block_b, D), lambda i: (i, 0)),
          full(F, D),
          full(1, D),
          full(D, H),
          full(D, H),
          full(1, H),
          full(H, D),
          full(1, D),
      ],
      out_specs=pl.BlockSpec((block_b, D), lambda i: (i, 0)),
      out_shape=jax.ShapeDtypeStruct((B, D), jnp.float32),
  )(features, id_repr, W_feat, b_feat, W1a, W1b, b1, W2, b2)


@jax.jit
def kernel(indices, features, table, W_feat, b_feat, W1, b1, W2, b2):
  V, D = table.shape
  B = indices.shape[0]
  idx = indices.astype(jnp.int32)
  id_repr = _make_sc_gather(V, D, B)(table, idx)
  W1a = W1[:D]
  W1b = W1[D:]
  return _tc_fused(features, id_repr, W_feat, b_feat.reshape(1, -1),
                   W1a, W1b, b1.reshape(1, -1), W2, b2.reshape(1, -1))


# speedup vs baseline: 1.7424x; 1.0371x over previous
import functools

import jax
import jax.numpy as jnp
from jax import lax
from jax.experimental import pallas as pl
from jax.experimental.pallas import tpu as pltpu
from jax.experimental.pallas import tpu_sc as plsc


_SC_INFO = plsc.get_sparse_core_info()
_NW = _SC_INFO.num_cores * _SC_INFO.num_subcores


@functools.lru_cache(maxsize=None)
def _make_sc_gather(V, D, B):
  b_per_w = B // _NW
  mesh = plsc.VectorSubcoreMesh(core_axis_name="c", subcore_axis_name="s")

  @functools.partial(
      pl.kernel,
      mesh=mesh,
      out_type=jax.ShapeDtypeStruct((B, D), jnp.float32),
      scratch_types=[
          pltpu.VMEM((b_per_w,), jnp.int32),
          pltpu.VMEM((b_per_w, D), jnp.float32),
          pltpu.SemaphoreType.DMA,
      ],
      name="sc_embedding_gather",
  )
  def gather_kernel(table_hbm, idx_hbm, out_hbm, idx_v, rows_v, sem):
    wid = lax.axis_index("s") * _SC_INFO.num_cores + lax.axis_index("c")
    base = wid * b_per_w
    pltpu.sync_copy(idx_hbm.at[pl.ds(base, b_per_w)], idx_v)
    pltpu.async_copy(table_hbm.at[idx_v], rows_v, sem).wait()
    pltpu.sync_copy(rows_v, out_hbm.at[pl.ds(base, b_per_w)])

  return gather_kernel


def _tc_fused_body(feat_ref, id_ref, wf_ref, bf_ref, w1a_ref, w1b_ref,
                   b1_ref, w2_ref, b2_ref, out_ref):
  idr = id_ref[...]
  fr = (jnp.dot(feat_ref[...], wf_ref[...], preferred_element_type=jnp.float32)
        + bf_ref[...])
  h = jnp.dot(idr, w1a_ref[...], preferred_element_type=jnp.float32)
  h += jnp.dot(fr, w1b_ref[...], preferred_element_type=jnp.float32)
  h = jnp.maximum(h + b1_ref[...], 0.0)
  g = jnp.dot(h, w2_ref[...], preferred_element_type=jnp.float32) + b2_ref[...]
  gate = jax.nn.sigmoid(g)
  out_ref[...] = gate * idr + (1.0 - gate) * fr


def _tc_fused(features, id_repr, W_feat, b_feat, W1a, W1b, b1, W2, b2,
              block_b=4096):
  B, F = features.shape
  D = id_repr.shape[1]
  H = W1a.shape[1]
  grid = (B // block_b,)
  full = lambda *s: pl.BlockSpec(s, lambda i: (0,) * len(s))
  return pl.pallas_call(
      _tc_fused_body,
      grid=grid,
      in_specs=[
          pl.BlockSpec((block_b, F), lambda i: (i, 0)),
          pl.BlockSpec((block_b, D), lambda i: (i, 0)),
          full(F, D),
          full(1, D),
          full(D, H),
          full(D, H),
          full(1, H),
          full(H, D),
          full(1, D),
      ],
      out_specs=pl.BlockSpec((block_b, D), lambda i: (i, 0)),
      out_shape=jax.ShapeDtypeStruct((B, D), jnp.float32),
  )(features, id_repr, W_feat, b_feat, W1a, W1b, b1, W2, b2)


@jax.jit
def kernel(indices, features, table, W_feat, b_feat, W1, b1, W2, b2):
  V, D = table.shape
  B = indices.shape[0]
  idx = indices.astype(jnp.int32)
  id_repr = _make_sc_gather(V, D, B)(table, idx)
  W1a = W1[:D]
  W1b = W1[D:]
  return _tc_fused(features, id_repr, W_feat, b_feat.reshape(1, -1),
                   W1a, W1b, b1.reshape(1, -1), W2, b2.reshape(1, -1))
